# baseline (speedup 1.0000x reference)
import functools
import math

import jax
import jax.numpy as jnp
from jax import lax
from jax.experimental import pallas as pl
from jax.experimental.pallas import tpu as pltpu
from jax.experimental.pallas import tpu_sc as plsc

MODEL_DIM = 128
SCALE = math.sqrt(float(MODEL_DIM))

NUM_CORES = 2
NUM_SUBCORES = 16
NUM_WORKERS = NUM_CORES * NUM_SUBCORES
LANES = 16

CHUNK = 128
NBUF = 5


@functools.cache
def _build(num_tokens: int, vocab: int):
    per_worker = num_tokens // NUM_WORKERS
    n_chunks = per_worker // CHUNK
    n_rounds = n_chunks // NBUF

    mesh = plsc.VectorSubcoreMesh(
        core_axis_name="c", subcore_axis_name="s",
        num_cores=NUM_CORES, num_subcores=NUM_SUBCORES,
    )

    n_chunks_pad = (n_chunks + 7) // 8 * 8

    scratch = (
        [pltpu.VMEM((n_chunks_pad, CHUNK), jnp.int32)]
        + [pltpu.VMEM((CHUNK, MODEL_DIM), jnp.float32) for _ in range(NBUF)]
        + [pltpu.SemaphoreType.DMA for _ in range(2 * NBUF)]
    )

    @functools.partial(
        pl.kernel,
        mesh=mesh,
        out_type=jax.ShapeDtypeStruct((num_tokens, MODEL_DIM), jnp.float32),
        scratch_types=scratch,
        compiler_params=pltpu.CompilerParams(use_tc_tiling_on_sc=True),
    )
    def emb_kernel(tokens_hbm, table_hbm, out_hbm, idx_v, *bufs_and_sems):
        bufs = bufs_and_sems[:NBUF]
        gsem = bufs_and_sems[NBUF:2 * NBUF]
        osem = bufs_and_sems[2 * NBUF:]

        wid = lax.axis_index("s") * NUM_CORES + lax.axis_index("c")
        base = wid * per_worker

        pltpu.sync_copy(tokens_hbm.at[wid], idx_v)

        def gather_desc(j, b, make):
            f = pltpu.make_async_copy if make else pltpu.async_copy
            return f(table_hbm.at[idx_v.at[j]], bufs[b], gsem[b])

        def write_desc(j, b, make):
            f = pltpu.make_async_copy if make else pltpu.async_copy
            return f(bufs[b], out_hbm.at[pl.ds(base + j * CHUNK, CHUNK)],
                     osem[b])

        def scale(b):
            def row_body(r, _):
                for c in range(MODEL_DIM // LANES):
                    sl = pl.ds(c * LANES, LANES)
                    bufs[b][r, sl] = bufs[b][r, sl] * SCALE
                return 0
            lax.fori_loop(0, CHUNK, row_body, 0, unroll=8)

        for b in range(NBUF):
            gather_desc(b, b, make=False)

        def round_body(g, _):
            j0 = g * NBUF
            for b in range(NBUF):
                gather_desc(j0 + b, b, make=True).wait()
                scale(b)
                write_desc(j0 + b, b, make=False)
            for b in range(NBUF):
                write_desc(j0 + b, b, make=True).wait()
                @pl.when(g < n_rounds - 1)
                def _():
                    gather_desc(j0 + NBUF + b, b, make=False)
            return 0

        lax.fori_loop(0, n_rounds, round_body, 0, unroll=False)

    return emb_kernel


def kernel(tokens, table):
    bsz, seq = tokens.shape
    num_tokens = bsz * seq
    per_worker = num_tokens // NUM_WORKERS
    n_chunks = per_worker // CHUNK
    n_chunks_pad = (n_chunks + 7) // 8 * 8
    idx = tokens.reshape(NUM_WORKERS, n_chunks, CHUNK).astype(jnp.int32)
    idx = jnp.pad(idx, ((0, 0), (0, n_chunks_pad - n_chunks), (0, 0)))
    out = _build(num_tokens, table.shape[0])(idx, table)
    return out.reshape(bsz, seq, MODEL_DIM)


# speedup vs baseline: 1.7787x; 1.7787x over previous
import functools
import math

import jax
import jax.numpy as jnp
from jax import lax
from jax.experimental import pallas as pl
from jax.experimental.pallas import tpu as pltpu
from jax.experimental.pallas import tpu_sc as plsc

MODEL_DIM = 128
SCALE = math.sqrt(float(MODEL_DIM))

NUM_CORES = 2
NUM_SUBCORES = 16
NUM_WORKERS = NUM_CORES * NUM_SUBCORES
LANES = 16

IDX_PAD = 128
NBUF = 8


@functools.cache
def _build(bsz: int, seq: int):
    per_worker = bsz // NUM_WORKERS
    n_rounds = per_worker // NBUF
    seq_pad = (seq + 7) // 8 * 8

    mesh = plsc.VectorSubcoreMesh(
        core_axis_name="c", subcore_axis_name="s",
        num_cores=NUM_CORES, num_subcores=NUM_SUBCORES,
    )

    scratch = (
        [pltpu.VMEM((per_worker, IDX_PAD), jnp.int32)]
        + [pltpu.VMEM((seq_pad, MODEL_DIM), jnp.float32) for _ in range(NBUF)]
        + [pltpu.SemaphoreType.DMA for _ in range(2 * NBUF)]
    )

    @functools.partial(
        pl.kernel,
        mesh=mesh,
        out_type=jax.ShapeDtypeStruct((bsz, seq, MODEL_DIM), jnp.float32),
        scratch_types=scratch,
        compiler_params=pltpu.CompilerParams(use_tc_tiling_on_sc=True),
    )
    def emb_kernel(tokens_hbm, table_hbm, out_hbm, idx_v, *bufs_and_sems):
        bufs = bufs_and_sems[:NBUF]
        gsem = bufs_and_sems[NBUF:2 * NBUF]
        osem = bufs_and_sems[2 * NBUF:]

        wid = lax.axis_index("s") * NUM_CORES + lax.axis_index("c")
        base = wid * per_worker

        pltpu.sync_copy(tokens_hbm.at[pl.ds(base, per_worker)], idx_v)

        def gather_desc(i, b, make):
            f = pltpu.make_async_copy if make else pltpu.async_copy
            return f(table_hbm.at[idx_v.at[i, pl.ds(0, seq)]],
                     bufs[b].at[pl.ds(0, seq)], gsem[b])

        def write_desc(i, b, make):
            f = pltpu.make_async_copy if make else pltpu.async_copy
            return f(bufs[b].at[pl.ds(0, seq)], out_hbm.at[base + i], osem[b])

        def scale(b):
            def row_body(r, _):
                for c in range(MODEL_DIM // LANES):
                    sl = pl.ds(c * LANES, LANES)
                    bufs[b][r, sl] = bufs[b][r, sl] * SCALE
                return 0
            lax.fori_loop(0, seq, row_body, 0, unroll=5)

        for b in range(NBUF):
            gather_desc(b, b, make=False)

        def round_body(g, _):
            i0 = g * NBUF
            for b in range(NBUF):
                gather_desc(i0 + b, b, make=True).wait()
                scale(b)
                write_desc(i0 + b, b, make=False)
            for b in range(NBUF):
                write_desc(i0 + b, b, make=True).wait()
                @pl.when(g < n_rounds - 1)
                def _():
                    gather_desc(i0 + NBUF + b, b, make=False)
            return 0

        lax.fori_loop(0, n_rounds, round_body, 0, unroll=False)

    return emb_kernel


def kernel(tokens, table):
    bsz, seq = tokens.shape
    idx = jnp.pad(tokens.astype(jnp.int32), ((0, 0), (0, IDX_PAD - seq)))
    return _build(bsz, seq)(idx, table)
